# baseline (speedup 1.0000x reference)
import functools

import jax
import jax.numpy as jnp
from jax import lax
from jax.experimental import pallas as pl
from jax.experimental.pallas import tpu as pltpu
from jax.experimental.pallas import tpu_sc as plsc

NC = 2
NS = 16
NW = NC * NS
LANES = 128


def _chunks(total, step):
    out = []
    s = 0
    while s < total:
        out.append((s, min(step, total - s)))
        s += step
    return out


def _sc_aggregate(bn, d, kch, x_hbm, src_hbm, dst_hbm, agg_out, deg_out,
                  src_v, dst_v, rows_v, sem0, sem1, shared_acc):
    nrow = shared_acc.shape[0]
    c = lax.axis_index("c")
    s = lax.axis_index("s")
    wid = s * NC + c

    rz = nrow // NS
    zbase = pl.multiple_of(s * rz, 8)
    ro = -(-(bn // NS) // 8) * 8
    obase = pl.multiple_of(jnp.minimum(s * ro, bn - ro), 8)
    ibase = wid * kch

    def _fill_rows(val):
        def _f(i, _):
            r = i // 8
            j = i % 8
            rows_v[r, pl.ds(j * 16, 16)] = jnp.full((16,), val, jnp.float32)
            return _
        lax.fori_loop(0, LANES * 8, _f, None)

    def _zero_acc():
        for (off, size) in _chunks(rz, LANES):
            pltpu.sync_copy(rows_v.at[pl.ds(0, size)],
                            shared_acc.at[pl.ds(zbase + off, size)])

    def _write_out(out_hbm):
        for (off, size) in _chunks(ro, LANES):
            pltpu.sync_copy(shared_acc.at[pl.ds(obase + off, size)],
                            rows_v.at[pl.ds(0, size)])
            pltpu.sync_copy(rows_v.at[pl.ds(0, size)],
                            out_hbm.at[c, pl.ds(obase + off, size)])

    _fill_rows(0.0)
    _zero_acc()
    plsc.subcore_barrier()

    def _gather_half(h, b):
        j, hh = h // 2, h % 2
        return pltpu.async_copy(
            x_hbm.at[src_v.at[j, pl.ds(64 * hh, 64)]],
            rows_v.at[pl.ds(64 * b, 64)], sem1 if b else sem0)

    def _outer1(kb, _):
        g8 = pl.multiple_of(ibase + kb * 8, 8)
        pltpu.sync_copy(src_hbm.at[pl.ds(g8, 8)], src_v)
        pltpu.sync_copy(dst_hbm.at[pl.ds(g8, 8)], dst_v)
        cp = _gather_half(0, 0)
        for h in range(16):
            b = h % 2
            cp.wait()
            if h < 15:
                cp = _gather_half(h + 1, 1 - b)
            j, hh = h // 2, h % 2
            pltpu.sync_copy(rows_v.at[pl.ds(64 * b, 64)],
                            shared_acc.at[dst_v.at[j, pl.ds(64 * hh, 64)]],
                            add=True)
        return _
    lax.fori_loop(0, kch // 8, _outer1, None)
    plsc.subcore_barrier()
    _write_out(agg_out)
    plsc.subcore_barrier()

    _fill_rows(0.0)
    _zero_acc()
    plsc.subcore_barrier()
    _fill_rows(1.0)

    def _outer2(kb, _):
        g8 = pl.multiple_of(ibase + kb * 8, 8)
        pltpu.sync_copy(dst_hbm.at[pl.ds(g8, 8)], dst_v)
        cps = [pltpu.async_copy(rows_v, shared_acc.at[dst_v.at[j]], sem0,
                                add=True) for j in range(8)]
        for cp in cps:
            cp.wait()
        return _
    lax.fori_loop(0, kch // 8, _outer2, None)
    plsc.subcore_barrier()
    _write_out(deg_out)


def _tc_body(x_ref, aggp_ref, degp_ref, wsT_ref, wnT_ref, b_ref, g_ref,
             be_ref, o_ref):
    agg = aggp_ref[0] + aggp_ref[1]
    deg = degp_ref[0, :, 0:1] + degp_ref[1, :, 0:1]
    neigh = agg / jnp.maximum(deg, 1.0)
    out = jnp.dot(x_ref[...], wsT_ref[...], preferred_element_type=jnp.float32)
    out = out + jnp.dot(neigh, wnT_ref[...],
                        preferred_element_type=jnp.float32)
    out = out + b_ref[...]
    mean = jnp.mean(out, axis=-1, keepdims=True)
    var = jnp.mean((out - mean) ** 2, axis=-1, keepdims=True)
    out = (out - mean) * lax.rsqrt(var + 1e-5) * g_ref[...] + be_ref[...]
    o_ref[...] = jnp.maximum(out, 0.0)


def kernel(x, edge_index, batch_size, W_self, W_neigh, bias, ln_gamma,
           ln_beta):
    B, N, D = x.shape
    bn = B * N
    dout = W_self.shape[0]

    src, dst = edge_index[0], edge_index[1]
    zero = jnp.asarray(batch_size, src.dtype) - B
    offsets = (jnp.arange(B, dtype=src.dtype)[:, None] + zero) * N
    src_e = (src[None, :] + offsets).reshape(-1).astype(jnp.int32)
    dst_e = (dst[None, :] + offsets).reshape(-1).astype(jnp.int32)
    e_tot = src_e.shape[0]

    e_pad = -(-e_tot // (NW * LANES * 8)) * (NW * LANES * 8)
    kch = e_pad // (NW * LANES)
    pad = e_pad - e_tot
    nrow = -(-(bn + 1) // (NS * 8)) * (NS * 8)
    pad_dst = bn + jnp.arange(pad, dtype=jnp.int32) % (nrow - bn)
    src_p = jnp.concatenate(
        [src_e, jnp.zeros((pad,), jnp.int32)]).reshape(-1, LANES)
    dst_p = jnp.concatenate([dst_e, pad_dst]).reshape(-1, LANES)

    x_flat = x.reshape(bn, D)

    mesh = plsc.VectorSubcoreMesh(core_axis_name="c", subcore_axis_name="s",
                                  num_cores=NC, num_subcores=NS)
    agg_part, deg_part = pl.kernel(
        functools.partial(_sc_aggregate, bn, D, kch),
        out_type=(
            jax.ShapeDtypeStruct((NC, bn, D), jnp.float32),
            jax.ShapeDtypeStruct((NC, bn, D), jnp.float32),
        ),
        mesh=mesh,
        scratch_types=[
            pltpu.VMEM((8, LANES), jnp.int32),
            pltpu.VMEM((8, LANES), jnp.int32),
            pltpu.VMEM((LANES, D), jnp.float32),
            pltpu.SemaphoreType.DMA,
            pltpu.SemaphoreType.DMA,
            pltpu.VMEM_SHARED((nrow, D), jnp.float32),
        ],
    )(x_flat, src_p, dst_p)

    r_blk = 1000
    grid = bn // r_blk
    out = pl.pallas_call(
        _tc_body,
        grid=(grid,),
        in_specs=[
            pl.BlockSpec((r_blk, D), lambda i: (i, 0)),
            pl.BlockSpec((NC, r_blk, D), lambda i: (0, i, 0)),
            pl.BlockSpec((NC, r_blk, D), lambda i: (0, i, 0)),
            pl.BlockSpec((D, dout), lambda i: (0, 0)),
            pl.BlockSpec((D, dout), lambda i: (0, 0)),
            pl.BlockSpec((1, dout), lambda i: (0, 0)),
            pl.BlockSpec((1, dout), lambda i: (0, 0)),
            pl.BlockSpec((1, dout), lambda i: (0, 0)),
        ],
        out_specs=pl.BlockSpec((r_blk, dout), lambda i: (i, 0)),
        out_shape=jax.ShapeDtypeStruct((bn, dout), jnp.float32),
    )(x_flat, agg_part, deg_part, W_self.T, W_neigh.T,
      bias.reshape(1, dout), ln_gamma.reshape(1, dout),
      ln_beta.reshape(1, dout))
    return out.reshape(B, N, dout)


# speedup vs baseline: 1.1633x; 1.1633x over previous
import functools

import jax
import jax.numpy as jnp
from jax import lax
from jax.experimental import pallas as pl
from jax.experimental.pallas import tpu as pltpu
from jax.experimental.pallas import tpu_sc as plsc

NC = 2
NS = 16
NW = NC * NS
LANES = 128


def _chunks(total, step):
    out = []
    s = 0
    while s < total:
        out.append((s, min(step, total - s)))
        s += step
    return out


def _sc_aggregate(bn, d, kch0, kch1, x_hbm, src_hbm, dst_hbm, agg_out,
                  deg_out, src_v, dst_v, rows_v, sem0, sem1, shared_acc):
    nrow = shared_acc.shape[0]
    c = lax.axis_index("c")
    s = lax.axis_index("s")
    wid = s * NC + c

    rz = nrow // NS
    zbase = pl.multiple_of(s * rz, 8)
    ro = -(-(bn // NS) // 8) * 8
    obase = pl.multiple_of(jnp.minimum(s * ro, bn - ro), 8)
    kchc = jnp.where(c == 0, kch0, kch1)
    ibase = c * (NS * kch0) + s * kchc

    def _fill_rows(val):
        def _f(i, _):
            r = i // 8
            j = i % 8
            rows_v[r, pl.ds(j * 16, 16)] = jnp.full((16,), val, jnp.float32)
            return _
        lax.fori_loop(0, LANES * 8, _f, None)

    def _zero_acc():
        for (off, size) in _chunks(rz, LANES):
            pltpu.sync_copy(rows_v.at[pl.ds(0, size)],
                            shared_acc.at[pl.ds(zbase + off, size)])

    def _write_out(out_hbm):
        for (off, size) in _chunks(ro, LANES):
            pltpu.sync_copy(shared_acc.at[pl.ds(obase + off, size)],
                            rows_v.at[pl.ds(0, size)])
            pltpu.sync_copy(rows_v.at[pl.ds(0, size)],
                            out_hbm.at[c, pl.ds(obase + off, size)])

    _fill_rows(0.0)
    _zero_acc()
    plsc.subcore_barrier()

    def _gather_half(h, b):
        j, hh = h // 2, h % 2
        return pltpu.async_copy(
            x_hbm.at[src_v.at[j, pl.ds(64 * hh, 64)]],
            rows_v.at[pl.ds(64 * b, 64)], sem1 if b else sem0)

    def _outer1(kb, _):
        g8 = pl.multiple_of(ibase + kb * 8, 8)
        pltpu.sync_copy(src_hbm.at[pl.ds(g8, 8)], src_v)
        pltpu.sync_copy(dst_hbm.at[pl.ds(g8, 8)], dst_v)
        cp = _gather_half(0, 0)
        for h in range(16):
            b = h % 2
            cp.wait()
            if h < 15:
                cp = _gather_half(h + 1, 1 - b)
            j, hh = h // 2, h % 2
            pltpu.sync_copy(rows_v.at[pl.ds(64 * b, 64)],
                            shared_acc.at[dst_v.at[j, pl.ds(64 * hh, 64)]],
                            add=True)
        return _
    lax.fori_loop(0, kchc // 8, _outer1, None)
    plsc.subcore_barrier()
    _write_out(agg_out)
    plsc.subcore_barrier()

    _fill_rows(0.0)
    _zero_acc()
    plsc.subcore_barrier()
    _fill_rows(1.0)

    def _outer2(kb, _):
        g8 = pl.multiple_of(ibase + kb * 8, 8)
        pltpu.sync_copy(dst_hbm.at[pl.ds(g8, 8)], dst_v)
        cps = [pltpu.async_copy(rows_v, shared_acc.at[dst_v.at[j]], sem0,
                                add=True) for j in range(8)]
        for cp in cps:
            cp.wait()
        return _
    lax.fori_loop(0, kchc // 8, _outer2, None)
    plsc.subcore_barrier()
    _write_out(deg_out)


def _tc_body(x_ref, aggp_ref, degp_ref, wsT_ref, wnT_ref, b_ref, g_ref,
             be_ref, o_ref):
    agg = aggp_ref[0] + aggp_ref[1]
    deg = degp_ref[0, :, 0:1] + degp_ref[1, :, 0:1]
    neigh = agg / jnp.maximum(deg, 1.0)
    out = jnp.dot(x_ref[...], wsT_ref[...], preferred_element_type=jnp.float32)
    out = out + jnp.dot(neigh, wnT_ref[...],
                        preferred_element_type=jnp.float32)
    out = out + b_ref[...]
    mean = jnp.mean(out, axis=-1, keepdims=True)
    var = jnp.mean((out - mean) ** 2, axis=-1, keepdims=True)
    out = (out - mean) * lax.rsqrt(var + 1e-5) * g_ref[...] + be_ref[...]
    o_ref[...] = jnp.maximum(out, 0.0)


def kernel(x, edge_index, batch_size, W_self, W_neigh, bias, ln_gamma,
           ln_beta):
    B, N, D = x.shape
    bn = B * N
    dout = W_self.shape[0]

    src, dst = edge_index[0], edge_index[1]
    zero = jnp.asarray(batch_size, src.dtype) - B
    offsets = (jnp.arange(B, dtype=src.dtype)[:, None] + zero) * N
    src_e = (src[None, :] + offsets).reshape(-1).astype(jnp.int32)
    dst_e = (dst[None, :] + offsets).reshape(-1).astype(jnp.int32)
    e_tot = src_e.shape[0]

    e_pad = -(-e_tot // (NW * LANES * 8)) * (NW * LANES * 8)
    kp = e_pad // (NS * LANES)
    kch0 = min(max(8, int(kp * 0.7) // 8 * 8), kp - 8)
    kch1 = kp - kch0
    pad = e_pad - e_tot
    nrow = -(-(bn + 1) // (NS * 8)) * (NS * 8)
    pad_dst = bn + jnp.arange(pad, dtype=jnp.int32) % (nrow - bn)
    src_p = jnp.concatenate(
        [src_e, jnp.zeros((pad,), jnp.int32)]).reshape(-1, LANES)
    dst_p = jnp.concatenate([dst_e, pad_dst]).reshape(-1, LANES)

    x_flat = x.reshape(bn, D)

    mesh = plsc.VectorSubcoreMesh(core_axis_name="c", subcore_axis_name="s",
                                  num_cores=NC, num_subcores=NS)
    agg_part, deg_part = pl.kernel(
        functools.partial(_sc_aggregate, bn, D, kch0, kch1),
        out_type=(
            jax.ShapeDtypeStruct((NC, bn, D), jnp.float32),
            jax.ShapeDtypeStruct((NC, bn, D), jnp.float32),
        ),
        mesh=mesh,
        scratch_types=[
            pltpu.VMEM((8, LANES), jnp.int32),
            pltpu.VMEM((8, LANES), jnp.int32),
            pltpu.VMEM((LANES, D), jnp.float32),
            pltpu.SemaphoreType.DMA,
            pltpu.SemaphoreType.DMA,
            pltpu.VMEM_SHARED((nrow, D), jnp.float32),
        ],
    )(x_flat, src_p, dst_p)

    r_blk = 1000
    grid = bn // r_blk
    out = pl.pallas_call(
        _tc_body,
        grid=(grid,),
        in_specs=[
            pl.BlockSpec((r_blk, D), lambda i: (i, 0)),
            pl.BlockSpec((NC, r_blk, D), lambda i: (0, i, 0)),
            pl.BlockSpec((NC, r_blk, D), lambda i: (0, i, 0)),
            pl.BlockSpec((D, dout), lambda i: (0, 0)),
            pl.BlockSpec((D, dout), lambda i: (0, 0)),
            pl.BlockSpec((1, dout), lambda i: (0, 0)),
            pl.BlockSpec((1, dout), lambda i: (0, 0)),
            pl.BlockSpec((1, dout), lambda i: (0, 0)),
        ],
        out_specs=pl.BlockSpec((r_blk, dout), lambda i: (i, 0)),
        out_shape=jax.ShapeDtypeStruct((bn, dout), jnp.float32),
    )(x_flat, agg_part, deg_part, W_self.T, W_neigh.T,
      bias.reshape(1, dout), ln_gamma.reshape(1, dout),
      ln_beta.reshape(1, dout))
    return out.reshape(B, N, dout)


# speedup vs baseline: 1.1984x; 1.0302x over previous
import functools

import jax
import jax.numpy as jnp
from jax import lax
from jax.experimental import pallas as pl
from jax.experimental.pallas import tpu as pltpu
from jax.experimental.pallas import tpu_sc as plsc

NC = 2
NS = 16
NW = NC * NS
LANES = 128


def _chunks(total, step):
    out = []
    s = 0
    while s < total:
        out.append((s, min(step, total - s)))
        s += step
    return out


def _sc_aggregate(bn, d, kch0, kch1, x_hbm, src_hbm, dst_hbm, agg_out,
                  deg_out, src_v, dst_v, rows_v, sem0, sem1, shared_acc):
    nrow = shared_acc.shape[0]
    c = lax.axis_index("c")
    s = lax.axis_index("s")
    wid = s * NC + c

    rz = nrow // NS
    zbase = pl.multiple_of(s * rz, 8)
    ro = -(-(bn // NS) // 8) * 8
    obase = pl.multiple_of(jnp.minimum(s * ro, bn - ro), 8)
    kchc = jnp.where(c == 0, kch0, kch1)
    ibase = c * (NS * kch0) + s * kchc

    def _fill_rows(val):
        def _f(i, _):
            r = i // 8
            j = i % 8
            rows_v[r, pl.ds(j * 16, 16)] = jnp.full((16,), val, jnp.float32)
            return _
        lax.fori_loop(0, LANES * 8, _f, None)

    def _zero_acc():
        for (off, size) in _chunks(rz, LANES):
            pltpu.sync_copy(rows_v.at[pl.ds(0, size)],
                            shared_acc.at[pl.ds(zbase + off, size)])

    def _write_out(out_hbm):
        for (off, size) in _chunks(ro, LANES):
            pltpu.sync_copy(shared_acc.at[pl.ds(obase + off, size)],
                            rows_v.at[pl.ds(0, size)])
            pltpu.sync_copy(rows_v.at[pl.ds(0, size)],
                            out_hbm.at[c, pl.ds(obase + off, size)])

    _fill_rows(0.0)
    _zero_acc()
    plsc.subcore_barrier()

    def _gather_half(h, b):
        j, hh = h // 2, h % 2
        return pltpu.async_copy(
            x_hbm.at[src_v.at[j, pl.ds(64 * hh, 64)]],
            rows_v.at[pl.ds(64 * b, 64)], sem1 if b else sem0)

    def _outer1(kb, _):
        g8 = pl.multiple_of(ibase + kb * 8, 8)
        pltpu.sync_copy(src_hbm.at[pl.ds(g8, 8)], src_v)
        pltpu.sync_copy(dst_hbm.at[pl.ds(g8, 8)], dst_v)
        cp = _gather_half(0, 0)
        for h in range(16):
            b = h % 2
            cp.wait()
            if h < 15:
                cp = _gather_half(h + 1, 1 - b)
            j, hh = h // 2, h % 2
            pltpu.sync_copy(rows_v.at[pl.ds(64 * b, 64)],
                            shared_acc.at[dst_v.at[j, pl.ds(64 * hh, 64)]],
                            add=True)
        return _
    lax.fori_loop(0, kchc // 8, _outer1, None)
    plsc.subcore_barrier()
    _write_out(agg_out)
    plsc.subcore_barrier()

    _fill_rows(0.0)
    _zero_acc()
    plsc.subcore_barrier()
    _fill_rows(1.0)

    def _outer2(kb, _):
        g8 = pl.multiple_of(ibase + kb * 8, 8)
        pltpu.sync_copy(dst_hbm.at[pl.ds(g8, 8)], dst_v)
        cps = [pltpu.async_copy(rows_v, shared_acc.at[dst_v.at[j]], sem0,
                                add=True) for j in range(8)]
        for cp in cps:
            cp.wait()
        return _
    lax.fori_loop(0, kchc // 8, _outer2, None)
    plsc.subcore_barrier()
    _write_out(deg_out)


def _tc_body(x_ref, aggp_ref, degp_ref, wsT_ref, wnT_ref, b_ref, g_ref,
             be_ref, o_ref):
    agg = aggp_ref[0] + aggp_ref[1]
    deg = degp_ref[0, :, 0:1] + degp_ref[1, :, 0:1]
    neigh = agg / jnp.maximum(deg, 1.0)
    out = jnp.dot(x_ref[...], wsT_ref[...], preferred_element_type=jnp.float32)
    out = out + jnp.dot(neigh, wnT_ref[...],
                        preferred_element_type=jnp.float32)
    out = out + b_ref[...]
    mean = jnp.mean(out, axis=-1, keepdims=True)
    var = jnp.mean((out - mean) ** 2, axis=-1, keepdims=True)
    out = (out - mean) * lax.rsqrt(var + 1e-5) * g_ref[...] + be_ref[...]
    o_ref[...] = jnp.maximum(out, 0.0)


def kernel(x, edge_index, batch_size, W_self, W_neigh, bias, ln_gamma,
           ln_beta):
    B, N, D = x.shape
    bn = B * N
    dout = W_self.shape[0]

    src, dst = edge_index[0], edge_index[1]
    zero = jnp.asarray(batch_size, src.dtype) - B
    offsets = (jnp.arange(B, dtype=src.dtype)[:, None] + zero) * N
    src_e = (src[None, :] + offsets).reshape(-1).astype(jnp.int32)
    dst_e = (dst[None, :] + offsets).reshape(-1).astype(jnp.int32)
    e_tot = src_e.shape[0]

    e_pad = -(-e_tot // (NW * LANES * 8)) * (NW * LANES * 8)
    kp = e_pad // (NS * LANES)
    kch0 = min(max(8, int(kp * 0.85) // 8 * 8), kp - 8)
    kch1 = kp - kch0
    pad = e_pad - e_tot
    nrow = -(-(bn + 1) // (NS * 8)) * (NS * 8)
    pad_dst = bn + jnp.arange(pad, dtype=jnp.int32) % (nrow - bn)
    src_p = jnp.concatenate(
        [src_e, jnp.zeros((pad,), jnp.int32)]).reshape(-1, LANES)
    dst_p = jnp.concatenate([dst_e, pad_dst]).reshape(-1, LANES)

    x_flat = x.reshape(bn, D)

    mesh = plsc.VectorSubcoreMesh(core_axis_name="c", subcore_axis_name="s",
                                  num_cores=NC, num_subcores=NS)
    agg_part, deg_part = pl.kernel(
        functools.partial(_sc_aggregate, bn, D, kch0, kch1),
        out_type=(
            jax.ShapeDtypeStruct((NC, bn, D), jnp.float32),
            jax.ShapeDtypeStruct((NC, bn, D), jnp.float32),
        ),
        mesh=mesh,
        scratch_types=[
            pltpu.VMEM((8, LANES), jnp.int32),
            pltpu.VMEM((8, LANES), jnp.int32),
            pltpu.VMEM((LANES, D), jnp.float32),
            pltpu.SemaphoreType.DMA,
            pltpu.SemaphoreType.DMA,
            pltpu.VMEM_SHARED((nrow, D), jnp.float32),
        ],
    )(x_flat, src_p, dst_p)

    r_blk = 1000
    grid = bn // r_blk
    out = pl.pallas_call(
        _tc_body,
        grid=(grid,),
        in_specs=[
            pl.BlockSpec((r_blk, D), lambda i: (i, 0)),
            pl.BlockSpec((NC, r_blk, D), lambda i: (0, i, 0)),
            pl.BlockSpec((NC, r_blk, D), lambda i: (0, i, 0)),
            pl.BlockSpec((D, dout), lambda i: (0, 0)),
            pl.BlockSpec((D, dout), lambda i: (0, 0)),
            pl.BlockSpec((1, dout), lambda i: (0, 0)),
            pl.BlockSpec((1, dout), lambda i: (0, 0)),
            pl.BlockSpec((1, dout), lambda i: (0, 0)),
        ],
        out_specs=pl.BlockSpec((r_blk, dout), lambda i: (i, 0)),
        out_shape=jax.ShapeDtypeStruct((bn, dout), jnp.float32),
    )(x_flat, agg_part, deg_part, W_self.T, W_neigh.T,
      bias.reshape(1, dout), ln_gamma.reshape(1, dout),
      ln_beta.reshape(1, dout))
    return out.reshape(B, N, dout)
